# Initial kernel scaffold; baseline (speedup 1.0000x reference)
import jax
import jax.numpy as jnp
from jax.experimental import pallas as pl


def kernel(inputs):
    raise NotImplementedError("write your pallas kernel here")


# speedup vs baseline: 12.2438x; 12.2438x over previous
import numpy as np
import jax
import jax.numpy as jnp
from jax.experimental import pallas as pl
from jax.experimental.pallas import tpu as pltpu

_H = 512
_PERC_T = 0.59593
_KS = (3, 5)
_ORDER = [0, 5, 1, 6, 2, 7, 3, 8, 4, 9]


def _resize_weights(n_in, n_out):
    x = (np.arange(n_out) + 0.5) * (n_in / n_out) - 0.5
    j = np.arange(n_in)
    w = np.maximum(0.0, 1.0 - np.abs(j[None, :] - x[:, None]))
    return (w / w.sum(1, keepdims=True)).astype(np.float32)


_WH = _resize_weights(2, 128)
_WW = _resize_weights(5, 128)
_BASIS = np.stack([np.outer(_WH[:, p // 5], _WW[:, p % 5]) for p in range(10)])


def _roll(a, s, axis):
    return pltpu.roll(a, s % a.shape[axis], axis)


def _block_sum(a, k, hm, wm):
    r = jnp.zeros_like(a)
    for d in range(-(k - 1), k):
        ok = (hm + d >= 0) & (hm + d <= k - 1)
        r = r + jnp.where(ok, _roll(a, -d, 0), 0.0)
    out = jnp.zeros_like(a)
    for d in range(-(k - 1), k):
        ok = (wm + d >= 0) & (wm + d <= k - 1)
        out = out + jnp.where(ok, _roll(r, -d, 1), 0.0)
    return out


def _metrics_for_k(xc, k, pad_ref):
    rows = -(-_H // k)
    hp = rows * k
    pt = (hp - _H) // 2
    p_cnt = float(rows * rows)
    kk = k * k

    pad_ref[:hp, :hp] = jnp.zeros((hp, hp), jnp.float32)
    pad_ref[pt:pt + _H, pt:pt + _H] = xc
    xp = pad_ref[:hp, :hp]

    hm = jax.lax.broadcasted_iota(jnp.int32, (hp, hp), 0) % k
    wm = jax.lax.broadcasted_iota(jnp.int32, (hp, hp), 1) % k

    t = jnp.zeros_like(xp)
    for r in range(k):
        t = jnp.where(hm == r, _roll(xp, r - k // 2, 0), t)
    center = jnp.zeros_like(xp)
    for r in range(k):
        center = jnp.where(wm == r, _roll(t, r - k // 2, 1), center)

    m = jnp.abs(xp - center) * 255.0 <= float(k * 8)
    mf = jnp.where(m, 1.0, 0.0)

    nb = _block_sum(mf, k, hm, wm)

    big = kk + 2
    init = hm * k + wm + 1
    lab0 = jnp.where(m, init, 0)
    wall_u = hm == 0
    wall_d = hm == k - 1
    wall_l = wm == 0
    wall_r = wm == k - 1

    def cc_body(_, lab):
        l = jnp.where(m, lab, big)
        up = jnp.where(wall_u, big, _roll(l, 1, 0))
        dn = jnp.where(wall_d, big, _roll(l, -1, 0))
        lf = jnp.where(wall_l, big, _roll(l, 1, 1))
        rt = jnp.where(wall_r, big, _roll(l, -1, 1))
        nl = jnp.minimum(l, jnp.minimum(jnp.minimum(up, dn),
                                        jnp.minimum(lf, rt)))
        return jnp.where(m, nl, 0)

    lab = jax.lax.fori_loop(0, kk, cc_body, lab0)

    s_root = jnp.sum(jnp.where(m & (lab == init), 1.0, 0.0))
    acn = jnp.floor(s_root / p_cnt)

    s_perc = jnp.sum(jnp.where(nb / float(kk) >= _PERC_T, 1.0, 0.0))
    perc = jnp.floor(s_perc / float(kk) / p_cnt)

    def area_body(j, amax):
        eq = jnp.where(lab == j, 1.0, 0.0)
        return jnp.maximum(amax, _block_sum(eq, k, hm, wm))

    amax = jax.lax.fori_loop(0, kk + 1, area_body, jnp.zeros_like(mf))
    ama = jnp.floor(jnp.sum(amax) / float(kk) / p_cnt)

    def hist_body(v, acc):
        fd_a, m1_a, m2_a = acc
        cnt = jnp.sum(jnp.where(nb == v.astype(jnp.float32), 1.0, 0.0))
        prob = (cnt / float(kk)) / p_cnt
        r = (v + 1).astype(jnp.float32)
        return (fd_a + prob / r, m1_a + prob * r, m2_a + prob * prob * r)

    fd, m1, m2 = jax.lax.fori_loop(
        0, kk, hist_body, (jnp.float32(0.0), jnp.float32(0.0),
                           jnp.float32(0.0)))
    lac = (m2 - m1 * m1) / (m1 * m1)
    return [acn, perc, ama, lac, fd]


def _fractal_kernel(x_ref, basis_ref, o_ref, pad_ref):
    for c in range(3):
        xc = x_ref[0, c, :, :]
        mets = []
        for k in _KS:
            mets.extend(_metrics_for_k(xc, k, pad_ref))
        acc = jnp.zeros((128, 128), jnp.float32)
        for p in range(10):
            acc = acc + mets[_ORDER[p]] * basis_ref[p, :, :]
        o_ref[0, c, :, :] = acc


def kernel(inputs):
    x = jnp.transpose(inputs, (0, 3, 1, 2))
    basis = jnp.asarray(_BASIS)
    out = pl.pallas_call(
        _fractal_kernel,
        grid=(x.shape[0],),
        in_specs=[
            pl.BlockSpec((1, 3, _H, _H), lambda b: (b, 0, 0, 0)),
            pl.BlockSpec((10, 128, 128), lambda b: (0, 0, 0)),
        ],
        out_specs=pl.BlockSpec((1, 3, 128, 128), lambda b: (b, 0, 0, 0)),
        out_shape=jax.ShapeDtypeStruct((x.shape[0], 3, 128, 128),
                                       jnp.float32),
        scratch_shapes=[pltpu.VMEM((515, 515), jnp.float32)],
    )(x, basis)
    return jnp.transpose(out, (0, 2, 3, 1))


# speedup vs baseline: 35.6049x; 2.9080x over previous
import numpy as np
import jax
import jax.numpy as jnp
from jax.experimental import pallas as pl
from jax.experimental.pallas import tpu as pltpu

_H = 512
_PERC_T = 0.59593
_KS = (3, 5)
_ORDER = [0, 5, 1, 6, 2, 7, 3, 8, 4, 9]


def _resize_weights(n_in, n_out):
    x = (np.arange(n_out) + 0.5) * (n_in / n_out) - 0.5
    j = np.arange(n_in)
    w = np.maximum(0.0, 1.0 - np.abs(j[None, :] - x[:, None]))
    return (w / w.sum(1, keepdims=True)).astype(np.float32)


_WH = _resize_weights(2, 128)
_WW = _resize_weights(5, 128)
_BASIS = np.stack([np.outer(_WH[:, p // 5], _WW[:, p % 5]) for p in range(10)])


def _planes(x, k):
    rows = -(-_H // k)
    hp = rows * k
    pt = (hp - _H) // 2
    xp = jnp.pad(x, ((0, 0), (pt, hp - _H - pt), (pt, hp - _H - pt), (0, 0)))
    xq = xp.reshape(x.shape[0], rows, k, rows, k, 3)
    xq = xq.transpose(0, 5, 2, 4, 1, 3)
    return xq.reshape(x.shape[0], 3 * k * k, rows, rows)


def _metrics_for_k(x_ref, c, k):
    rows = -(-_H // k)
    p_cnt = float(rows * rows)
    kk = k * k
    big = kk + 2

    xs = [x_ref[0, c * kk + p, :, :] for p in range(kk)]
    ctr = xs[(k // 2) * k + (k // 2)]
    m = [jnp.abs(xs[p] - ctr) * 255.0 <= float(k * 8) for p in range(kk)]
    mf = [jnp.where(m[p], 1.0, 0.0) for p in range(kk)]

    nb = mf[0]
    for p in range(1, kk):
        nb = nb + mf[p]

    neigh = []
    for p in range(kk):
        dy, dx = p // k, p % k
        ns = []
        if dy > 0:
            ns.append(p - k)
        if dy < k - 1:
            ns.append(p + k)
        if dx > 0:
            ns.append(p - 1)
        if dx < k - 1:
            ns.append(p + 1)
        neigh.append(tuple(ns))

    lab0 = tuple(jnp.where(m[p], p + 1, 0) for p in range(kk))

    def cc_body(_, lab):
        l = [jnp.where(m[p], lab[p], big) for p in range(kk)]
        out = []
        for p in range(kk):
            nl = l[p]
            for q in neigh[p]:
                nl = jnp.minimum(nl, l[q])
            out.append(jnp.where(m[p], nl, 0))
        return tuple(out)

    lab = jax.lax.fori_loop(0, kk, cc_body, lab0)

    s_root = jnp.float32(0.0)
    for p in range(kk):
        s_root = s_root + jnp.sum(
            jnp.where(m[p] & (lab[p] == p + 1), 1.0, 0.0))
    acn = jnp.floor(s_root / p_cnt)

    s_perc = jnp.sum(jnp.where(nb / float(kk) >= _PERC_T, 1.0, 0.0))
    perc = jnp.floor(s_perc / p_cnt)

    def area_body(j, amax):
        cnt = jnp.where(lab[0] == j, 1.0, 0.0)
        for p in range(1, kk):
            cnt = cnt + jnp.where(lab[p] == j, 1.0, 0.0)
        return jnp.maximum(amax, cnt)

    amax = jax.lax.fori_loop(0, kk + 1, area_body,
                             jnp.zeros((rows, rows), jnp.float32))
    ama = jnp.floor(jnp.sum(amax) / p_cnt)

    def hist_body(v, acc):
        fd_a, m1_a, m2_a = acc
        cnt = jnp.sum(jnp.where(nb == v.astype(jnp.float32), 1.0, 0.0))
        prob = cnt / p_cnt
        r = (v + 1).astype(jnp.float32)
        return (fd_a + prob / r, m1_a + prob * r, m2_a + prob * prob * r)

    fd, m1, m2 = jax.lax.fori_loop(
        0, kk, hist_body, (jnp.float32(0.0), jnp.float32(0.0),
                           jnp.float32(0.0)))
    lac = (m2 - m1 * m1) / (m1 * m1)
    return [acn, perc, ama, lac, fd]


def _fractal_kernel(x3_ref, x5_ref, basis_ref, o_ref):
    refs = {3: x3_ref, 5: x5_ref}
    for c in range(3):
        mets = []
        for k in _KS:
            mets.extend(_metrics_for_k(refs[k], c, k))
        acc = jnp.zeros((128, 128), jnp.float32)
        for p in range(10):
            acc = acc + mets[_ORDER[p]] * basis_ref[p, :, :]
        o_ref[0, c, :, :] = acc


def kernel(inputs):
    b = inputs.shape[0]
    x3 = _planes(inputs, 3)
    x5 = _planes(inputs, 5)
    basis = jnp.asarray(_BASIS)
    out = pl.pallas_call(
        _fractal_kernel,
        grid=(b,),
        in_specs=[
            pl.BlockSpec((1,) + x3.shape[1:], lambda i: (i, 0, 0, 0)),
            pl.BlockSpec((1,) + x5.shape[1:], lambda i: (i, 0, 0, 0)),
            pl.BlockSpec((10, 128, 128), lambda i: (0, 0, 0)),
        ],
        out_specs=pl.BlockSpec((1, 3, 128, 128), lambda i: (i, 0, 0, 0)),
        out_shape=jax.ShapeDtypeStruct((b, 3, 128, 128), jnp.float32),
    )(x3, x5, basis)
    return jnp.transpose(out, (0, 2, 3, 1))
